# baseline (speedup 1.0000x reference)
import functools

import jax
import jax.numpy as jnp
from jax import lax
from jax.experimental import pallas as pl
from jax.experimental.pallas import tpu as pltpu
from jax.experimental.pallas import tpu_sc as plsc

_NUM_POS = 8192
_EMB = 1024
_CHUNK = 63
_NBUF = 2


@functools.cache
def _copy_kernel():
    info = plsc.get_sparse_core_info()
    nc, ns = info.num_cores, info.num_subcores
    nw = nc * ns
    rows_per_w = _NUM_POS // nw
    chunks = []
    rem = rows_per_w
    while rem > 0:
        c = min(rem, _CHUNK)
        chunks.append(c)
        rem -= c
    mesh = plsc.VectorSubcoreMesh(core_axis_name="c", subcore_axis_name="s")

    @functools.partial(
        pl.kernel,
        mesh=mesh,
        out_type=jax.ShapeDtypeStruct((_NUM_POS, _EMB), jnp.float32),
        compiler_params=pltpu.CompilerParams(use_tc_tiling_on_sc=False),
        scratch_types=[
            pltpu.VMEM((_NBUF, _CHUNK, _EMB), jnp.float32),
            pltpu.SemaphoreType.DMA,
            pltpu.SemaphoreType.DMA,
            pltpu.SemaphoreType.DMA,
            pltpu.SemaphoreType.DMA,
        ],
    )
    def copy_k(table_hbm, out_hbm, buf, si0, si1, so0, so1):
        sin = (si0, si1)
        sout = (so0, so1)
        sid = lax.axis_index("s")
        wid = sid * nc + lax.axis_index("c")
        base = wid * rows_per_w
        offs = [sum(chunks[:i]) for i in range(len(chunks))]
        n = len(chunks)
        hin = [None] * n
        hout = [None] * n

        def start_in(i):
            b = i % _NBUF
            if i >= _NBUF:
                hout[i - _NBUF].wait()
            hin[i] = pltpu.async_copy(
                table_hbm.at[pl.ds(base + offs[i], chunks[i])],
                buf.at[b, pl.ds(0, chunks[i])], sin[b])

        start_in(0)
        for i in range(n):
            if i + 1 < n:
                start_in(i + 1)
            b = i % _NBUF
            hin[i].wait()
            hout[i] = pltpu.async_copy(
                buf.at[b, pl.ds(0, chunks[i])],
                out_hbm.at[pl.ds(base + offs[i], chunks[i])], sout[b])
        for i in range(max(n - _NBUF, 0), n):
            hout[i].wait()

    return copy_k


def kernel(x, pos_embed_weight):
    del x
    out = _copy_kernel()(pos_embed_weight)
    return out[None]


# speedup vs baseline: 2.5040x; 2.5040x over previous
import functools

import jax
import jax.numpy as jnp
from jax import lax
from jax.experimental import pallas as pl
from jax.experimental.pallas import tpu as pltpu
from jax.experimental.pallas import tpu_sc as plsc

_NUM_POS = 8192
_EMB = 1024
_CHUNK = 56
_NBUF = 2


@functools.cache
def _copy_kernel():
    info = plsc.get_sparse_core_info()
    nc, ns = info.num_cores, info.num_subcores
    nw = nc * ns
    rows_per_w = _NUM_POS // nw
    chunks = []
    rem = rows_per_w
    while rem > 0:
        c = min(rem, _CHUNK)
        chunks.append(c)
        rem -= c
    mesh = plsc.VectorSubcoreMesh(core_axis_name="c", subcore_axis_name="s")

    @functools.partial(
        pl.kernel,
        mesh=mesh,
        out_type=jax.ShapeDtypeStruct((_NUM_POS, _EMB), jnp.float32),
        scratch_types=[
            pltpu.VMEM((_NBUF, _CHUNK, _EMB), jnp.float32),
            pltpu.SemaphoreType.DMA,
            pltpu.SemaphoreType.DMA,
            pltpu.SemaphoreType.DMA,
            pltpu.SemaphoreType.DMA,
        ],
    )
    def copy_k(table_hbm, out_hbm, buf, si0, si1, so0, so1):
        sin = (si0, si1)
        sout = (so0, so1)
        sid = lax.axis_index("s")
        wid = sid * nc + lax.axis_index("c")
        base = wid * rows_per_w
        offs = [sum(chunks[:i]) for i in range(len(chunks))]
        n = len(chunks)
        hin = [None] * n
        hout = [None] * n

        def start_in(i):
            b = i % _NBUF
            if i >= _NBUF:
                hout[i - _NBUF].wait()
            hin[i] = pltpu.async_copy(
                table_hbm.at[pl.ds(base + offs[i], chunks[i])],
                buf.at[b, pl.ds(0, chunks[i])], sin[b])

        start_in(0)
        for i in range(n):
            if i + 1 < n:
                start_in(i + 1)
            b = i % _NBUF
            hin[i].wait()
            hout[i] = pltpu.async_copy(
                buf.at[b, pl.ds(0, chunks[i])],
                out_hbm.at[pl.ds(base + offs[i], chunks[i])], sout[b])
        for i in range(max(n - _NBUF, 0), n):
            hout[i].wait()

    return copy_k


def kernel(x, pos_embed_weight):
    del x
    out = _copy_kernel()(pos_embed_weight)
    return out[None]
